# Initial kernel scaffold; baseline (speedup 1.0000x reference)
import jax
import jax.numpy as jnp
from jax.experimental import pallas as pl


def kernel(x, edge_index, edge_type, basis1, comp1, root1, basis2, comp2, root2):
    raise NotImplementedError("write your pallas kernel here")


# speedup vs baseline: 57.4979x; 57.4979x over previous
import functools

import jax
import jax.numpy as jnp
from jax import lax
from jax.experimental import pallas as pl
from jax.experimental.pallas import tpu as pltpu
from jax.experimental.pallas import tpu_sc as plsc

_N = 10000
_E = 160000
_DIN = 128
_HID = 32
_OUT = 16
_R = 8
_NR = _N * _R

_NC = 2
_NS = 16
_NW = _NC * _NS
_EW = _E // _NW
_T = 1000
_NCH = _EW // _T
_TP = 1008
_NPT = _N // _NS
_CPT = _NR // _NS
_CPTP = 5008

_BN = 2000
_NBLK = _N // _BN

_mesh = plsc.VectorSubcoreMesh(core_axis_name="c", subcore_axis_name="s")
_sc_params = pltpu.CompilerParams(use_tc_tiling_on_sc=False)


def _zero_1d(ref, n):
    z = jnp.zeros((16,), jnp.float32)

    @pl.loop(0, n, step=16)
    def _(i):
        ref[pl.ds(i, 16)] = z


@functools.partial(
    pl.kernel,
    mesh=_mesh,
    out_type=jax.ShapeDtypeStruct((_NC * _NR,), jnp.float32),
    compiler_params=_sc_params,
    scratch_types=[
        pltpu.VMEM((1, _T), jnp.int32),
        pltpu.VMEM((_TP,), jnp.float32),
        pltpu.VMEM((_CPTP,), jnp.float32),
        pltpu.VMEM_SHARED((_NR,), jnp.float32),
        pltpu.SemaphoreType.DMA,
    ],
)
def _sc_count(k_hbm, out_hbm, k_v, ones_v, zbuf, cnt_sh, sem):
    cid = lax.axis_index("c")
    sid = lax.axis_index("s")
    wid = sid * _NC + cid

    _zero_1d(zbuf, _CPTP)
    one = jnp.full((16,), 1.0, jnp.float32)

    @pl.loop(0, _TP, step=16)
    def _(i):
        ones_v[pl.ds(i, 16)] = one

    pltpu.sync_copy(zbuf.at[pl.ds(0, _CPT)], cnt_sh.at[pl.ds(sid * _CPT, _CPT)])
    plsc.subcore_barrier()

    @pl.loop(0, _NCH)
    def _(ch):
        base = pl.multiple_of(wid * _EW + ch * _T, 8)
        pltpu.sync_copy(k_hbm.at[pl.ds(base, _T)], k_v.at[0])
        pltpu.sync_copy(ones_v.at[pl.ds(0, _T)], cnt_sh.at[k_v.at[0]], add=True)

    plsc.subcore_barrier()
    pltpu.sync_copy(cnt_sh.at[pl.ds(sid * _CPT, _CPT)], zbuf.at[pl.ds(0, _CPT)])
    pltpu.sync_copy(zbuf.at[pl.ds(0, _CPT)],
                    out_hbm.at[pl.ds(cid * _NR + sid * _CPT, _CPT)])


def _make_sc_agg(F):
    @functools.partial(
        pl.kernel,
        mesh=_mesh,
        out_type=jax.ShapeDtypeStruct((_NC * _N, F), jnp.float32),
        compiler_params=_sc_params,
        scratch_types=[
            pltpu.VMEM((1, _T), jnp.int32),
            pltpu.VMEM((1, _T), jnp.int32),
            pltpu.VMEM((1, _T), jnp.int32),
            pltpu.VMEM((_T, F), jnp.float32),
            pltpu.VMEM((_TP,), jnp.float32),
            pltpu.VMEM((_TP,), jnp.float32),
            pltpu.VMEM_SHARED((_N, F), jnp.float32),
            pltpu.SemaphoreType.DMA,
            pltpu.SemaphoreType.DMA,
        ],
    )
    def _sc_agg(y_hbm, g_hbm, d_hbm, k_hbm, cnt_hbm, out_hbm,
                g_v, d_v, k_v, rows_v, c_v, w_v, agg_sh, sem1, sem2):
        cid = lax.axis_index("c")
        sid = lax.axis_index("s")
        wid = sid * _NC + cid

        @pl.loop(0, _NPT)
        def _(i):
            for f0 in range(0, F, 16):
                rows_v[i, pl.ds(f0, 16)] = jnp.zeros((16,), jnp.float32)

        pltpu.sync_copy(rows_v.at[pl.ds(0, _NPT)],
                        agg_sh.at[pl.ds(sid * _NPT, _NPT)])
        plsc.subcore_barrier()

        @pl.loop(0, _NCH)
        def _(ch):
            base = pl.multiple_of(wid * _EW + ch * _T, 8)
            pltpu.sync_copy(g_hbm.at[pl.ds(base, _T)], g_v.at[0])
            pltpu.sync_copy(d_hbm.at[pl.ds(base, _T)], d_v.at[0])
            pltpu.sync_copy(k_hbm.at[pl.ds(base, _T)], k_v.at[0])
            cp1 = pltpu.async_copy(y_hbm.at[g_v.at[0]], rows_v, sem1)
            cp2 = pltpu.async_copy(cnt_hbm.at[k_v.at[0]], c_v.at[pl.ds(0, _T)],
                                   sem2)
            cp2.wait()

            @pl.loop(0, _TP, step=16)
            def _(i):
                w_v[pl.ds(i, 16)] = 1.0 / c_v[pl.ds(i, 16)]

            cp1.wait()

            @pl.loop(0, _T, step=16)
            def _(i):
                wv = w_v[pl.ds(i, 16)]
                for j in range(16):
                    wj = wv[j]
                    for f0 in range(0, F, 16):
                        rows_v[i + j, pl.ds(f0, 16)] = (
                            rows_v[i + j, pl.ds(f0, 16)] * wj)

            pltpu.sync_copy(rows_v, agg_sh.at[d_v.at[0]], add=True)

        plsc.subcore_barrier()

        @pl.when(sid < _N // _T)
        def _():
            pltpu.sync_copy(agg_sh.at[pl.ds(sid * _T, _T)], rows_v)
            pltpu.sync_copy(rows_v, out_hbm.at[pl.ds(cid * _N + sid * _T, _T)])

    return _sc_agg


_sc_agg1 = _make_sc_agg(_HID)
_sc_agg2 = _make_sc_agg(_OUT)


def _weights_body(c1_ref, b1_ref, c2_ref, b2_ref, w1_ref, w2_ref):
    w1_ref[...] = jnp.dot(c1_ref[...], b1_ref[...],
                          preferred_element_type=jnp.float32)
    w2_ref[...] = jnp.dot(c2_ref[...], b2_ref[...],
                          preferred_element_type=jnp.float32)


_weights_call = pl.pallas_call(
    _weights_body,
    out_shape=(
        jax.ShapeDtypeStruct((_R, _DIN * _HID), jnp.float32),
        jax.ShapeDtypeStruct((_R, _HID * _OUT), jnp.float32),
    ),
)


def _dense1_body(x_ref, w_ref, root_ref, cntp_ref, y_ref, xr_ref, cnt_ref):
    i = pl.program_id(0)
    j = pl.program_id(1)
    xb = x_ref[...]
    y_ref[0] = jnp.dot(xb, w_ref[0], preferred_element_type=jnp.float32)

    @pl.when(j == 0)
    def _():
        xr_ref[...] = jnp.dot(xb, root_ref[...],
                              preferred_element_type=jnp.float32)

    @pl.when((i == 0) & (j == 0))
    def _():
        cnt_ref[...] = cntp_ref[0] + cntp_ref[1]


_dense1_call = pl.pallas_call(
    _dense1_body,
    grid=(_NBLK, _R),
    in_specs=[
        pl.BlockSpec((_BN, _DIN), lambda i, j: (i, 0)),
        pl.BlockSpec((1, _DIN, _HID), lambda i, j: (j, 0, 0)),
        pl.BlockSpec((_DIN, _HID), lambda i, j: (0, 0)),
        pl.BlockSpec((2, 625, 128), lambda i, j: (0, 0, 0)),
    ],
    out_specs=[
        pl.BlockSpec((1, _BN, _HID), lambda i, j: (j, i, 0)),
        pl.BlockSpec((_BN, _HID), lambda i, j: (i, 0)),
        pl.BlockSpec((625, 128), lambda i, j: (0, 0)),
    ],
    out_shape=(
        jax.ShapeDtypeStruct((_R, _N, _HID), jnp.float32),
        jax.ShapeDtypeStruct((_N, _HID), jnp.float32),
        jax.ShapeDtypeStruct((625, 128), jnp.float32),
    ),
)


def _dense2_body(aggp_ref, xr_ref, w_ref, root_ref, y_ref, xr2_ref):
    j = pl.program_id(1)
    h = jax.nn.relu(aggp_ref[0] + aggp_ref[1] + xr_ref[...])
    y_ref[0] = jnp.dot(h, w_ref[0], preferred_element_type=jnp.float32)

    @pl.when(j == 0)
    def _():
        xr2_ref[...] = jnp.dot(h, root_ref[...],
                               preferred_element_type=jnp.float32)


_dense2_call = pl.pallas_call(
    _dense2_body,
    grid=(_NBLK, _R),
    in_specs=[
        pl.BlockSpec((2, _BN, _HID), lambda i, j: (0, i, 0)),
        pl.BlockSpec((_BN, _HID), lambda i, j: (i, 0)),
        pl.BlockSpec((1, _HID, _OUT), lambda i, j: (j, 0, 0)),
        pl.BlockSpec((_HID, _OUT), lambda i, j: (0, 0)),
    ],
    out_specs=[
        pl.BlockSpec((1, _BN, _OUT), lambda i, j: (j, i, 0)),
        pl.BlockSpec((_BN, _OUT), lambda i, j: (i, 0)),
    ],
    out_shape=(
        jax.ShapeDtypeStruct((_R, _N, _OUT), jnp.float32),
        jax.ShapeDtypeStruct((_N, _OUT), jnp.float32),
    ),
)


def _final_body(aggp_ref, xr_ref, z_ref):
    z_ref[...] = aggp_ref[0] + aggp_ref[1] + xr_ref[...]


_final_call = pl.pallas_call(
    _final_body,
    grid=(_NBLK,),
    in_specs=[
        pl.BlockSpec((2, _BN, _OUT), lambda i: (0, i, 0)),
        pl.BlockSpec((_BN, _OUT), lambda i: (i, 0)),
    ],
    out_specs=pl.BlockSpec((_BN, _OUT), lambda i: (i, 0)),
    out_shape=jax.ShapeDtypeStruct((_N, _OUT), jnp.float32),
)


def kernel(x, edge_index, edge_type, basis1, comp1, root1,
           basis2, comp2, root2):
    src = edge_index[0].astype(jnp.int32)
    dst = edge_index[1].astype(jnp.int32)
    et = edge_type.astype(jnp.int32)
    g = et * _N + src
    k = dst * _R + et

    w1cat, w2cat = _weights_call(
        comp1, basis1.reshape(30, _DIN * _HID),
        comp2, basis2.reshape(30, _HID * _OUT))
    w1 = w1cat.reshape(_R, _DIN, _HID)
    w2 = w2cat.reshape(_R, _HID, _OUT)

    cntp = _sc_count(k)
    y1, xr1, cnt2d = _dense1_call(x, w1, root1,
                                  cntp.reshape(2, 625, 128))
    cnt = cnt2d.reshape(_NR)

    aggp1 = _sc_agg1(y1.reshape(_R * _N, _HID), g, dst, k, cnt)
    y2, xr2 = _dense2_call(aggp1.reshape(2, _N, _HID), xr1, w2, root2)
    aggp2 = _sc_agg2(y2.reshape(_R * _N, _OUT), g, dst, k, cnt)
    return _final_call(aggp2.reshape(2, _N, _OUT), xr2)


# speedup vs baseline: 60.3631x; 1.0498x over previous
import functools

import jax
import jax.numpy as jnp
from jax import lax
from jax.experimental import pallas as pl
from jax.experimental.pallas import tpu as pltpu
from jax.experimental.pallas import tpu_sc as plsc

_N = 10000
_E = 160000
_DIN = 128
_HID = 32
_OUT = 16
_R = 8
_NR = _N * _R

_NC = 2
_NS = 16
_NW = _NC * _NS
_EW = _E // _NW
_T = 1000
_NCH = _EW // _T
_TP = 1008
_NPT = _N // _NS
_CPT = _NR // _NS
_CPTP = 5008

_BN = 2000
_NBLK = _N // _BN

_mesh = plsc.VectorSubcoreMesh(core_axis_name="c", subcore_axis_name="s")
_sc_params = pltpu.CompilerParams(use_tc_tiling_on_sc=False)


_EPT = _E // _NS
_NCH1 = _EPT // _T


def _scale_rows(rows, w_ref, F):
    @pl.loop(0, _T, step=16)
    def _(i):
        wv = w_ref[pl.ds(i, 16)]
        for j in range(16):
            wj = wv[j]
            for f0 in range(0, F, 16):
                rows[i + j, pl.ds(f0, 16)] = rows[i + j, pl.ds(f0, 16)] * wj


def _zero_rows(rows, n, F):
    @pl.loop(0, n)
    def _(i):
        for f0 in range(0, F, 16):
            rows[i, pl.ds(f0, 16)] = jnp.zeros((16,), jnp.float32)


def _fill_1d(ref, n, val):
    v = jnp.full((16,), val, jnp.float32)

    @pl.loop(0, n, step=16)
    def _(i):
        ref[pl.ds(i, 16)] = v


@functools.partial(
    pl.kernel,
    mesh=_mesh,
    out_type=[
        jax.ShapeDtypeStruct((_NC * _N, _HID), jnp.float32),
        jax.ShapeDtypeStruct((_E,), jnp.float32),
    ],
    compiler_params=_sc_params,
    scratch_types=[
        pltpu.VMEM((2, _T), jnp.int32),
        pltpu.VMEM((2, _T), jnp.int32),
        pltpu.VMEM((2, _T), jnp.int32),
        pltpu.VMEM((2, _T, _HID), jnp.float32),
        pltpu.VMEM((2, _TP), jnp.float32),
        pltpu.VMEM((_TP,), jnp.float32),
        pltpu.VMEM((_TP,), jnp.float32),
        pltpu.VMEM((_CPTP,), jnp.float32),
        pltpu.VMEM_SHARED((_NR,), jnp.float32),
        pltpu.VMEM_SHARED((_N, _HID), jnp.float32),
        pltpu.SemaphoreType.DMA,
        pltpu.SemaphoreType.DMA,
        pltpu.SemaphoreType.DMA,
        pltpu.SemaphoreType.DMA,
        pltpu.SemaphoreType.DMA,
        pltpu.SemaphoreType.DMA,
    ],
)
def _sc_agg1(y_hbm, g_hbm, d_hbm, k_hbm, agg_out, w_out,
             g_v, d_v, k_v, rows_v, c_v, w_v, ones_v, zbuf, cnt_sh, agg_sh,
             sg0, sg1, sc0, sc1, ss0, ss1):
    cid = lax.axis_index("c")
    sid = lax.axis_index("s")
    wid = sid * _NC + cid
    sgs = (sg0, sg1)
    scs = (sc0, sc1)
    sss = (ss0, ss1)

    _fill_1d(ones_v, _TP, 1.0)
    _fill_1d(zbuf, _CPTP, 0.0)
    _zero_rows(rows_v.at[0], _NPT, _HID)
    pltpu.sync_copy(zbuf.at[pl.ds(0, _CPT)],
                    cnt_sh.at[pl.ds(sid * _CPT, _CPT)])
    pltpu.sync_copy(rows_v.at[0].at[pl.ds(0, _NPT)],
                    agg_sh.at[pl.ds(sid * _NPT, _NPT)])
    plsc.subcore_barrier()

    adds = [None, None]

    def _load_k1(ch, b):
        base = pl.multiple_of(sid * _EPT + ch * _T, 8)
        pltpu.sync_copy(k_hbm.at[pl.ds(base, _T)], k_v.at[b])

    _load_k1(0, 0)
    for ch in range(_NCH1):
        b = ch & 1
        nb = b ^ 1
        if ch + 1 < _NCH1:
            if ch >= 1:
                adds[nb].wait()
            _load_k1(ch + 1, nb)
        adds[b] = pltpu.async_copy(ones_v.at[pl.ds(0, _T)],
                                   cnt_sh.at[k_v.at[b]], sgs[b], add=True)
    adds[0].wait()
    adds[1].wait()
    plsc.subcore_barrier()

    gts = [None, None]
    cts = [None, None]
    sts = [None, None]

    def _load_idx(ch, b):
        base = pl.multiple_of(wid * _EW + ch * _T, 8)
        pltpu.sync_copy(g_hbm.at[pl.ds(base, _T)], g_v.at[b])
        pltpu.sync_copy(d_hbm.at[pl.ds(base, _T)], d_v.at[b])
        pltpu.sync_copy(k_hbm.at[pl.ds(base, _T)], k_v.at[b])

    def _start_gathers(b):
        gts[b] = pltpu.async_copy(y_hbm.at[g_v.at[b]], rows_v.at[b], sgs[b])
        cts[b] = pltpu.async_copy(cnt_sh.at[k_v.at[b]],
                                  c_v.at[b].at[pl.ds(0, _T)], scs[b])

    _load_idx(0, 0)
    _start_gathers(0)
    for ch in range(_NCH):
        b = ch & 1
        nb = b ^ 1
        if ch + 1 < _NCH:
            if ch >= 1:
                sts[nb].wait()
            _load_idx(ch + 1, nb)
            _start_gathers(nb)
        cts[b].wait()

        @pl.loop(0, _TP, step=16)
        def _(i, _b=b):
            w_v[pl.ds(i, 16)] = 1.0 / c_v[_b, pl.ds(i, 16)]

        wbase = pl.multiple_of(wid * _EW + ch * _T, 8)
        pltpu.sync_copy(w_v.at[pl.ds(0, _T)], w_out.at[pl.ds(wbase, _T)])
        gts[b].wait()
        _scale_rows(rows_v.at[b], w_v, _HID)
        sts[b] = pltpu.async_copy(rows_v.at[b], agg_sh.at[d_v.at[b]],
                                  sss[b], add=True)
    sts[0].wait()
    sts[1].wait()
    plsc.subcore_barrier()

    @pl.when(sid < _N // _T)
    def _():
        pltpu.sync_copy(agg_sh.at[pl.ds(sid * _T, _T)], rows_v.at[0])
        pltpu.sync_copy(rows_v.at[0],
                        agg_out.at[pl.ds(cid * _N + sid * _T, _T)])


@functools.partial(
    pl.kernel,
    mesh=_mesh,
    out_type=jax.ShapeDtypeStruct((_NC * _N, _OUT), jnp.float32),
    compiler_params=_sc_params,
    scratch_types=[
        pltpu.VMEM((2, _T), jnp.int32),
        pltpu.VMEM((2, _T), jnp.int32),
        pltpu.VMEM((2, _TP), jnp.float32),
        pltpu.VMEM((2, _T, _OUT), jnp.float32),
        pltpu.VMEM_SHARED((_N, _OUT), jnp.float32),
        pltpu.SemaphoreType.DMA,
        pltpu.SemaphoreType.DMA,
        pltpu.SemaphoreType.DMA,
        pltpu.SemaphoreType.DMA,
    ],
)
def _sc_agg2(y_hbm, g_hbm, d_hbm, w_hbm, agg_out,
             g_v, d_v, w_v, rows_v, agg_sh, sg0, sg1, ss0, ss1):
    cid = lax.axis_index("c")
    sid = lax.axis_index("s")
    wid = sid * _NC + cid
    sgs = (sg0, sg1)
    sss = (ss0, ss1)

    _zero_rows(rows_v.at[0], _NPT, _OUT)
    pltpu.sync_copy(rows_v.at[0].at[pl.ds(0, _NPT)],
                    agg_sh.at[pl.ds(sid * _NPT, _NPT)])
    plsc.subcore_barrier()

    gts = [None, None]
    sts = [None, None]

    def _load_idx(ch, b):
        base = pl.multiple_of(wid * _EW + ch * _T, 8)
        pltpu.sync_copy(g_hbm.at[pl.ds(base, _T)], g_v.at[b])
        pltpu.sync_copy(d_hbm.at[pl.ds(base, _T)], d_v.at[b])
        pltpu.sync_copy(w_hbm.at[pl.ds(base, _T)], w_v.at[b].at[pl.ds(0, _T)])

    def _start_gather(b):
        gts[b] = pltpu.async_copy(y_hbm.at[g_v.at[b]], rows_v.at[b], sgs[b])

    _load_idx(0, 0)
    _start_gather(0)
    for ch in range(_NCH):
        b = ch & 1
        nb = b ^ 1
        if ch + 1 < _NCH:
            if ch >= 1:
                sts[nb].wait()
            _load_idx(ch + 1, nb)
            _start_gather(nb)
        gts[b].wait()
        _scale_rows(rows_v.at[b], w_v.at[b], _OUT)
        sts[b] = pltpu.async_copy(rows_v.at[b], agg_sh.at[d_v.at[b]],
                                  sss[b], add=True)
    sts[0].wait()
    sts[1].wait()
    plsc.subcore_barrier()

    @pl.when(sid < _N // _T)
    def _():
        pltpu.sync_copy(agg_sh.at[pl.ds(sid * _T, _T)], rows_v.at[0])
        pltpu.sync_copy(rows_v.at[0],
                        agg_out.at[pl.ds(cid * _N + sid * _T, _T)])


def _weights_body(c1_ref, b1_ref, c2_ref, b2_ref, w1_ref, w2_ref):
    w1_ref[...] = jnp.dot(c1_ref[...], b1_ref[...],
                          preferred_element_type=jnp.float32)
    w2_ref[...] = jnp.dot(c2_ref[...], b2_ref[...],
                          preferred_element_type=jnp.float32)


_weights_call = pl.pallas_call(
    _weights_body,
    out_shape=(
        jax.ShapeDtypeStruct((_R, _DIN * _HID), jnp.float32),
        jax.ShapeDtypeStruct((_R, _HID * _OUT), jnp.float32),
    ),
)


def _dense1_body(x_ref, w_ref, root_ref, y_ref, xr_ref):
    j = pl.program_id(1)
    xb = x_ref[...]
    y_ref[0] = jnp.dot(xb, w_ref[0], preferred_element_type=jnp.float32)

    @pl.when(j == 0)
    def _():
        xr_ref[...] = jnp.dot(xb, root_ref[...],
                              preferred_element_type=jnp.float32)


_dense1_call = pl.pallas_call(
    _dense1_body,
    grid=(_NBLK, _R),
    in_specs=[
        pl.BlockSpec((_BN, _DIN), lambda i, j: (i, 0)),
        pl.BlockSpec((1, _DIN, _HID), lambda i, j: (j, 0, 0)),
        pl.BlockSpec((_DIN, _HID), lambda i, j: (0, 0)),
    ],
    out_specs=[
        pl.BlockSpec((1, _BN, _HID), lambda i, j: (j, i, 0)),
        pl.BlockSpec((_BN, _HID), lambda i, j: (i, 0)),
    ],
    out_shape=(
        jax.ShapeDtypeStruct((_R, _N, _HID), jnp.float32),
        jax.ShapeDtypeStruct((_N, _HID), jnp.float32),
    ),
)


def _dense2_body(aggp_ref, xr_ref, w_ref, root_ref, y_ref, xr2_ref):
    j = pl.program_id(1)
    h = jax.nn.relu(aggp_ref[0] + aggp_ref[1] + xr_ref[...])
    y_ref[0] = jnp.dot(h, w_ref[0], preferred_element_type=jnp.float32)

    @pl.when(j == 0)
    def _():
        xr2_ref[...] = jnp.dot(h, root_ref[...],
                               preferred_element_type=jnp.float32)


_dense2_call = pl.pallas_call(
    _dense2_body,
    grid=(_NBLK, _R),
    in_specs=[
        pl.BlockSpec((2, _BN, _HID), lambda i, j: (0, i, 0)),
        pl.BlockSpec((_BN, _HID), lambda i, j: (i, 0)),
        pl.BlockSpec((1, _HID, _OUT), lambda i, j: (j, 0, 0)),
        pl.BlockSpec((_HID, _OUT), lambda i, j: (0, 0)),
    ],
    out_specs=[
        pl.BlockSpec((1, _BN, _OUT), lambda i, j: (j, i, 0)),
        pl.BlockSpec((_BN, _OUT), lambda i, j: (i, 0)),
    ],
    out_shape=(
        jax.ShapeDtypeStruct((_R, _N, _OUT), jnp.float32),
        jax.ShapeDtypeStruct((_N, _OUT), jnp.float32),
    ),
)


def _final_body(aggp_ref, xr_ref, z_ref):
    z_ref[...] = aggp_ref[0] + aggp_ref[1] + xr_ref[...]


_final_call = pl.pallas_call(
    _final_body,
    grid=(_NBLK,),
    in_specs=[
        pl.BlockSpec((2, _BN, _OUT), lambda i: (0, i, 0)),
        pl.BlockSpec((_BN, _OUT), lambda i: (i, 0)),
    ],
    out_specs=pl.BlockSpec((_BN, _OUT), lambda i: (i, 0)),
    out_shape=jax.ShapeDtypeStruct((_N, _OUT), jnp.float32),
)


def kernel(x, edge_index, edge_type, basis1, comp1, root1,
           basis2, comp2, root2):
    src = edge_index[0].astype(jnp.int32)
    dst = edge_index[1].astype(jnp.int32)
    et = edge_type.astype(jnp.int32)
    g = et * _N + src
    k = dst * _R + et

    w1cat, w2cat = _weights_call(
        comp1, basis1.reshape(30, _DIN * _HID),
        comp2, basis2.reshape(30, _HID * _OUT))
    w1 = w1cat.reshape(_R, _DIN, _HID)
    w2 = w2cat.reshape(_R, _HID, _OUT)

    y1, xr1 = _dense1_call(x, w1, root1)
    aggp1, w_edge = _sc_agg1(y1.reshape(_R * _N, _HID), g, dst, k)
    y2, xr2 = _dense2_call(aggp1.reshape(2, _N, _HID), xr1, w2, root2)
    aggp2 = _sc_agg2(y2.reshape(_R * _N, _OUT), g, dst, w_edge)
    return _final_call(aggp2.reshape(2, _N, _OUT), xr2)


# speedup vs baseline: 94.8270x; 1.5709x over previous
import functools

import jax
import jax.numpy as jnp
from jax import lax
from jax.experimental import pallas as pl
from jax.experimental.pallas import tpu as pltpu
from jax.experimental.pallas import tpu_sc as plsc

_N = 10000
_E = 160000
_DIN = 128
_HID = 32
_OUT = 16
_R = 8
_NR = _N * _R

_NC = 2
_NS = 16
_NW = _NC * _NS
_EW = _E // _NW
_T = 1000
_NCH = _EW // _T
_TP = 1008
_NPT = _N // _NS
_CPT = _NR // _NS
_CPTP = 5008

_BN = 2000
_NBLK = _N // _BN

_mesh = plsc.VectorSubcoreMesh(core_axis_name="c", subcore_axis_name="s")
_sc_params = pltpu.CompilerParams(use_tc_tiling_on_sc=False)


_EPT = _E // _NS
_NCH1 = _EPT // _T


@functools.partial(
    pl.kernel,
    mesh=_mesh,
    out_type=jax.ShapeDtypeStruct((_NC * _NR,), jnp.float32),
    compiler_params=_sc_params,
    scratch_types=[
        pltpu.VMEM((1, _T), jnp.int32),
        pltpu.VMEM((_TP,), jnp.float32),
        pltpu.VMEM((_CPTP,), jnp.float32),
        pltpu.VMEM_SHARED((_NR,), jnp.float32),
        pltpu.SemaphoreType.DMA,
    ],
)
def _sc_count(k_hbm, out_hbm, k_v, ones_v, zbuf, cnt_sh, sem):
    cid = lax.axis_index("c")
    sid = lax.axis_index("s")
    wid = sid * _NC + cid

    _fill_1d(zbuf, _CPTP, 0.0)
    _fill_1d(ones_v, _TP, 1.0)
    pltpu.sync_copy(zbuf.at[pl.ds(0, _CPT)], cnt_sh.at[pl.ds(sid * _CPT, _CPT)])
    plsc.subcore_barrier()

    @pl.loop(0, _NCH)
    def _(ch):
        base = pl.multiple_of(wid * _EW + ch * _T, 8)
        pltpu.sync_copy(k_hbm.at[pl.ds(base, _T)], k_v.at[0])
        pltpu.sync_copy(ones_v.at[pl.ds(0, _T)], cnt_sh.at[k_v.at[0]], add=True)

    plsc.subcore_barrier()
    pltpu.sync_copy(cnt_sh.at[pl.ds(sid * _CPT, _CPT)], zbuf.at[pl.ds(0, _CPT)])
    pltpu.sync_copy(zbuf.at[pl.ds(0, _CPT)],
                    out_hbm.at[pl.ds(cid * _NR + sid * _CPT, _CPT)])


def _scale_rows(rows, w_ref, F):
    @pl.loop(0, _T, step=16)
    def _(i):
        wv = w_ref[pl.ds(i, 16)]
        for j in range(16):
            wj = wv[j]
            for f0 in range(0, F, 16):
                rows[i + j, pl.ds(f0, 16)] = rows[i + j, pl.ds(f0, 16)] * wj


def _zero_rows(rows, n, F):
    @pl.loop(0, n)
    def _(i):
        for f0 in range(0, F, 16):
            rows[i, pl.ds(f0, 16)] = jnp.zeros((16,), jnp.float32)


def _fill_1d(ref, n, val):
    v = jnp.full((16,), val, jnp.float32)

    @pl.loop(0, n, step=16)
    def _(i):
        ref[pl.ds(i, 16)] = v


@functools.partial(
    pl.kernel,
    mesh=_mesh,
    out_type=[
        jax.ShapeDtypeStruct((_NC * _N, _HID), jnp.float32),
        jax.ShapeDtypeStruct((_E,), jnp.float32),
    ],
    compiler_params=_sc_params,
    scratch_types=[
        pltpu.VMEM((2, _T), jnp.int32),
        pltpu.VMEM((2, _T), jnp.int32),
        pltpu.VMEM((2, _T), jnp.int32),
        pltpu.VMEM((2, _T, _HID), jnp.float32),
        pltpu.VMEM((2, _TP), jnp.float32),
        pltpu.VMEM((_TP,), jnp.float32),
        pltpu.VMEM_SHARED((_N, _HID), jnp.float32),
        pltpu.SemaphoreType.DMA,
        pltpu.SemaphoreType.DMA,
        pltpu.SemaphoreType.DMA,
        pltpu.SemaphoreType.DMA,
        pltpu.SemaphoreType.DMA,
        pltpu.SemaphoreType.DMA,
    ],
)
def _sc_agg1(y_hbm, g_hbm, d_hbm, k_hbm, cnt_hbm, agg_out, w_out,
             g_v, d_v, k_v, rows_v, c_v, w_v, agg_sh,
             sg0, sg1, sc0, sc1, ss0, ss1):
    cid = lax.axis_index("c")
    sid = lax.axis_index("s")
    wid = sid * _NC + cid
    sgs = (sg0, sg1)
    scs = (sc0, sc1)
    sss = (ss0, ss1)

    _zero_rows(rows_v.at[0], _NPT, _HID)
    pltpu.sync_copy(rows_v.at[0].at[pl.ds(0, _NPT)],
                    agg_sh.at[pl.ds(sid * _NPT, _NPT)])
    plsc.subcore_barrier()

    @pl.loop(0, _NCH)
    def _(ch):
        base = pl.multiple_of(wid * _EW + ch * _T, 8)
        pltpu.sync_copy(g_hbm.at[pl.ds(base, _T)], g_v.at[0])
        pltpu.sync_copy(d_hbm.at[pl.ds(base, _T)], d_v.at[0])
        pltpu.sync_copy(k_hbm.at[pl.ds(base, _T)], k_v.at[0])
        gt = pltpu.async_copy(y_hbm.at[g_v.at[0]], rows_v.at[0], sg0)
        ct = pltpu.async_copy(cnt_hbm.at[k_v.at[0]],
                              c_v.at[0].at[pl.ds(0, _T)], sc0)
        ct.wait()

        @pl.loop(0, _TP, step=16)
        def _(i):
            w_v[pl.ds(i, 16)] = 1.0 / c_v[0, pl.ds(i, 16)]

        pltpu.sync_copy(w_v.at[pl.ds(0, _T)], w_out.at[pl.ds(base, _T)])
        gt.wait()
        _scale_rows(rows_v.at[0], w_v, _HID)
        pltpu.sync_copy(rows_v.at[0], agg_sh.at[d_v.at[0]], add=True)

    plsc.subcore_barrier()

    @pl.when(sid < _N // _T)
    def _():
        pltpu.sync_copy(agg_sh.at[pl.ds(sid * _T, _T)], rows_v.at[0])
        pltpu.sync_copy(rows_v.at[0],
                        agg_out.at[pl.ds(cid * _N + sid * _T, _T)])


@functools.partial(
    pl.kernel,
    mesh=_mesh,
    out_type=jax.ShapeDtypeStruct((_NC * _N, _OUT), jnp.float32),
    compiler_params=_sc_params,
    scratch_types=[
        pltpu.VMEM((2, _T), jnp.int32),
        pltpu.VMEM((2, _T), jnp.int32),
        pltpu.VMEM((2, _TP), jnp.float32),
        pltpu.VMEM((2, _T, _OUT), jnp.float32),
        pltpu.VMEM_SHARED((_N, _OUT), jnp.float32),
        pltpu.SemaphoreType.DMA,
        pltpu.SemaphoreType.DMA,
        pltpu.SemaphoreType.DMA,
        pltpu.SemaphoreType.DMA,
    ],
)
def _sc_agg2(y_hbm, g_hbm, d_hbm, w_hbm, agg_out,
             g_v, d_v, w_v, rows_v, agg_sh, sg0, sg1, ss0, ss1):
    cid = lax.axis_index("c")
    sid = lax.axis_index("s")
    wid = sid * _NC + cid
    sgs = (sg0, sg1)
    sss = (ss0, ss1)

    _zero_rows(rows_v.at[0], _NPT, _OUT)
    pltpu.sync_copy(rows_v.at[0].at[pl.ds(0, _NPT)],
                    agg_sh.at[pl.ds(sid * _NPT, _NPT)])
    plsc.subcore_barrier()

    @pl.loop(0, _NCH)
    def _(ch):
        base = pl.multiple_of(wid * _EW + ch * _T, 8)
        pltpu.sync_copy(g_hbm.at[pl.ds(base, _T)], g_v.at[0])
        pltpu.sync_copy(d_hbm.at[pl.ds(base, _T)], d_v.at[0])
        pltpu.sync_copy(w_hbm.at[pl.ds(base, _T)], w_v.at[0].at[pl.ds(0, _T)])
        gt = pltpu.async_copy(y_hbm.at[g_v.at[0]], rows_v.at[0], sg0)
        gt.wait()
        _scale_rows(rows_v.at[0], w_v.at[0], _OUT)
        pltpu.sync_copy(rows_v.at[0], agg_sh.at[d_v.at[0]], add=True)

    plsc.subcore_barrier()

    @pl.when(sid < _N // _T)
    def _():
        pltpu.sync_copy(agg_sh.at[pl.ds(sid * _T, _T)], rows_v.at[0])
        pltpu.sync_copy(rows_v.at[0],
                        agg_out.at[pl.ds(cid * _N + sid * _T, _T)])


def _weights_body(c1_ref, b1_ref, c2_ref, b2_ref, w1_ref, w2_ref):
    w1_ref[...] = jnp.dot(c1_ref[...], b1_ref[...],
                          preferred_element_type=jnp.float32)
    w2_ref[...] = jnp.dot(c2_ref[...], b2_ref[...],
                          preferred_element_type=jnp.float32)


_weights_call = pl.pallas_call(
    _weights_body,
    out_shape=(
        jax.ShapeDtypeStruct((_R, _DIN * _HID), jnp.float32),
        jax.ShapeDtypeStruct((_R, _HID * _OUT), jnp.float32),
    ),
)


def _dense1_body(x_ref, w_ref, root_ref, cntp_ref, y_ref, xr_ref, cnt_ref):
    i = pl.program_id(0)
    q = pl.program_id(1)
    xb = x_ref[...]
    y_ref[...] = jnp.dot(xb, w_ref[0], preferred_element_type=jnp.float32)

    @pl.when(q == 0)
    def _():
        xr_ref[...] = jnp.dot(xb, root_ref[...],
                              preferred_element_type=jnp.float32)

    @pl.when((i == 0) & (q == 0))
    def _():
        cnt_ref[...] = cntp_ref[0] + cntp_ref[1]


_dense1_call = pl.pallas_call(
    _dense1_body,
    grid=(_NBLK, 2),
    in_specs=[
        pl.BlockSpec((_BN, _DIN), lambda i, q: (i, 0)),
        pl.BlockSpec((1, _DIN, 128), lambda i, q: (q, 0, 0)),
        pl.BlockSpec((_DIN, _HID), lambda i, q: (0, 0)),
        pl.BlockSpec((2, 625, 128), lambda i, q: (0, 0, 0)),
    ],
    out_specs=[
        pl.BlockSpec((_BN, 128), lambda i, q: (q * _NBLK + i, 0)),
        pl.BlockSpec((_BN, _HID), lambda i, q: (i, 0)),
        pl.BlockSpec((625, 128), lambda i, q: (0, 0)),
    ],
    out_shape=(
        jax.ShapeDtypeStruct((2 * _N, 128), jnp.float32),
        jax.ShapeDtypeStruct((_N, _HID), jnp.float32),
        jax.ShapeDtypeStruct((625, 128), jnp.float32),
    ),
)


def _dense2_body(a0_ref, a1_ref, xr_ref, w_ref, root_ref, y_ref, xr2_ref):
    h = jax.nn.relu(a0_ref[...] + a1_ref[...] + xr_ref[...])
    y_ref[...] = jnp.dot(h, w_ref[...], preferred_element_type=jnp.float32)
    xr2_ref[...] = jnp.dot(h, root_ref[...], preferred_element_type=jnp.float32)


_dense2_call = pl.pallas_call(
    _dense2_body,
    grid=(_NBLK,),
    in_specs=[
        pl.BlockSpec((_BN, _HID), lambda i: (i, 0)),
        pl.BlockSpec((_BN, _HID), lambda i: (_NBLK + i, 0)),
        pl.BlockSpec((_BN, _HID), lambda i: (i, 0)),
        pl.BlockSpec((_HID, 128), lambda i: (0, 0)),
        pl.BlockSpec((_HID, _OUT), lambda i: (0, 0)),
    ],
    out_specs=[
        pl.BlockSpec((_BN, 128), lambda i: (i, 0)),
        pl.BlockSpec((_BN, _OUT), lambda i: (i, 0)),
    ],
    out_shape=(
        jax.ShapeDtypeStruct((_N, 128), jnp.float32),
        jax.ShapeDtypeStruct((_N, _OUT), jnp.float32),
    ),
)


def _final_body(a0_ref, a1_ref, xr_ref, z_ref):
    z_ref[...] = a0_ref[...] + a1_ref[...] + xr_ref[...]


_final_call = pl.pallas_call(
    _final_body,
    grid=(_NBLK,),
    in_specs=[
        pl.BlockSpec((_BN, _OUT), lambda i: (i, 0)),
        pl.BlockSpec((_BN, _OUT), lambda i: (_NBLK + i, 0)),
        pl.BlockSpec((_BN, _OUT), lambda i: (i, 0)),
    ],
    out_specs=pl.BlockSpec((_BN, _OUT), lambda i: (i, 0)),
    out_shape=jax.ShapeDtypeStruct((_N, _OUT), jnp.float32),
)


def kernel(x, edge_index, edge_type, basis1, comp1, root1,
           basis2, comp2, root2):
    src = edge_index[0].astype(jnp.int32)
    dst = edge_index[1].astype(jnp.int32)
    et = edge_type.astype(jnp.int32)
    g2 = (et // 4) * (4 * _N) + src * 4 + (et % 4)
    g3 = src * 8 + et
    k = dst * _R + et

    w1cat, w2cat = _weights_call(
        comp1, basis1.reshape(30, _DIN * _HID),
        comp2, basis2.reshape(30, _HID * _OUT))
    w1q = (w1cat.reshape(2, 4, _DIN, _HID)
           .transpose(0, 2, 1, 3).reshape(2, _DIN, 128))
    w2q = (w2cat.reshape(_R, _HID, _OUT)
           .transpose(1, 0, 2).reshape(_HID, 128))

    cntp = _sc_count(k)
    y1q, xr1, cnt2d = _dense1_call(x, w1q, root1, cntp.reshape(2, 625, 128))
    cnt = cnt2d.reshape(_NR)
    aggp1, w_edge = _sc_agg1(y1q.reshape(_R * _N, _HID), g2, dst, k, cnt)
    y2q, xr2 = _dense2_call(aggp1, aggp1, xr1, w2q, root2)
    aggp2 = _sc_agg2(y2q.reshape(_R * _N, _OUT), g3, dst, w_edge)
    return _final_call(aggp2, aggp2, xr2)
